# Initial kernel scaffold; baseline (speedup 1.0000x reference)
import jax
import jax.numpy as jnp
from jax.experimental import pallas as pl


def kernel(x, edge_index, batch, W1_rel, b1_rel, W1_root, W2_rel, b2_rel, W2_root, W3_rel, b3_rel, W3_root, W_lin, b_lin):
    raise NotImplementedError("write your pallas kernel here")


# speedup vs baseline: 19.3342x; 19.3342x over previous
import functools

import jax
import jax.numpy as jnp
from jax import lax
from jax.experimental import pallas as pl
from jax.experimental.pallas import tpu as pltpu
from jax.experimental.pallas import tpu_sc as plsc

_G = 64
_NPAD = 112
_CHUNK = 128
_NW = 32
_EALIGN = _NW * _CHUNK * 8


def _dotT(a, w):
    return lax.dot_general(a, w, (((1,), (1,)), ((), ())),
                           preferred_element_type=jnp.float32)


def _proj1_body(x_ref, wrel_ref, wroot_ref, p_ref, r_ref):
    n = x_ref.shape[0]
    x = x_ref[...]
    p_ref[0:n, :] = _dotT(x, wrel_ref[...])
    r_ref[0:n, :] = _dotT(x, wroot_ref[...])
    pad = p_ref.shape[0] - n
    p_ref[n:, :] = jnp.zeros((pad, p_ref.shape[1]), jnp.float32)
    r_ref[n:, :] = jnp.zeros((pad, r_ref.shape[1]), jnp.float32)


def _combine1_body(acc_ref, r_ref, b_ref, h_ref):
    n = r_ref.shape[0] - _NPAD
    h = jnp.maximum(acc_ref[0] + acc_ref[1] + r_ref[...] + b_ref[...], 0.0)
    h_ref[0:n, :] = h[0:n, :]
    h_ref[n:, :] = jnp.zeros((_NPAD, h_ref.shape[1]), jnp.float32)


def _combine_mm_body(acc_ref, h_ref, wrel_ref, wroot_ref, b_ref, out_ref):
    n = h_ref.shape[0] - _NPAD
    agg = acc_ref[0] + acc_ref[1]
    v = _dotT(agg, wrel_ref[...]) + _dotT(h_ref[...], wroot_ref[...])
    v = jnp.maximum(v + b_ref[...], 0.0)
    out_ref[0:n, :] = v[0:n, :]
    out_ref[n:, :] = jnp.zeros((_NPAD, out_ref.shape[1]), jnp.float32)


def _final_body(acc_ref, h_ref, wrel_ref, wroot_ref, b_ref, batch_ref,
                wlin_ref, blin_ref, out_ref):
    npd = h_ref.shape[0]
    agg = acc_ref[0] + acc_ref[1]
    h3 = _dotT(agg, wrel_ref[...]) + _dotT(h_ref[...], wroot_ref[...])
    h3 = jnp.maximum(h3 + b_ref[...], 0.0)
    seg = batch_ref[...]
    mt = (lax.broadcasted_iota(jnp.int32, (_G, npd), 0) == seg)
    pooled = lax.dot_general(mt.astype(jnp.float32), h3,
                             (((1,), (0,)), ((), ())),
                             preferred_element_type=jnp.float32)
    out_ref[...] = _dotT(pooled, wlin_ref[...]) + blin_ref[...]


def _tc_call(body, out_shapes, *args):
    return pl.pallas_call(
        body,
        out_shape=out_shapes,
    )(*args)


def _make_seg_sum(npd, d, ec):
    cpw = ec // _NW
    rpt = npd // 16
    mesh = plsc.VectorSubcoreMesh(core_axis_name="c", subcore_axis_name="s",
                                  num_cores=2, num_subcores=16)

    @functools.partial(
        pl.kernel,
        out_type=jax.ShapeDtypeStruct((2, npd, d), jnp.float32),
        mesh=mesh,
        compiler_params=pltpu.CompilerParams(use_tc_tiling_on_sc=False),
        scratch_types=[
            pltpu.VMEM((cpw, _CHUNK), jnp.int32),
            pltpu.VMEM((cpw, _CHUNK), jnp.int32),
            pltpu.VMEM((_CHUNK, d), jnp.float32),
            pltpu.VMEM_SHARED((npd, d), jnp.float32),
            pltpu.VMEM_SHARED((npd, d), jnp.float32),
            pltpu.SemaphoreType.DMA,
        ],
    )
    def seg(table, src, dst, zeros, out, sidx, didx, rows, tbl, acc, sem):
        c = lax.axis_index("c")
        s = lax.axis_index("s")
        w = c * 16 + s
        pltpu.sync_copy(src.at[pl.ds(w * cpw, cpw)], sidx)
        pltpu.sync_copy(dst.at[pl.ds(w * cpw, cpw)], didx)
        pltpu.sync_copy(table.at[pl.ds(s * rpt, rpt)],
                        tbl.at[pl.ds(s * rpt, rpt)])
        pltpu.sync_copy(zeros.at[pl.ds(s * rpt, rpt)],
                        acc.at[pl.ds(s * rpt, rpt)])
        plsc.subcore_barrier()

        def body(j, carry):
            pltpu.async_copy(tbl.at[sidx.at[j]], rows, sem).wait()
            pltpu.sync_copy(rows, acc.at[didx.at[j]], add=True)
            return carry

        lax.fori_loop(0, cpw, body, 0, unroll=False)
        plsc.subcore_barrier()
        pltpu.sync_copy(acc.at[pl.ds(s * rpt, rpt)],
                        out.at[c, pl.ds(s * rpt, rpt)])

    return seg


def kernel(x, edge_index, batch, W1_rel, b1_rel, W1_root, W2_rel, b2_rel,
           W2_root, W3_rel, b3_rel, W3_root, W_lin, b_lin):
    n, d_in = x.shape
    e = edge_index.shape[1]
    npd = n + _NPAD

    epad = -(-e // _EALIGN) * _EALIGN
    pad = epad - e
    src = jnp.concatenate([edge_index[0], jnp.full((pad,), n, jnp.int32)])
    dst = jnp.concatenate([edge_index[1], jnp.full((pad,), n, jnp.int32)])
    src2d = src.reshape(epad // _CHUNK, _CHUNK)
    dst2d = dst.reshape(epad // _CHUNK, _CHUNK)
    ec = epad // _CHUNK

    zeros16 = jnp.zeros((npd, 16), jnp.float32)
    zeros32 = jnp.zeros((npd, 32), jnp.float32)
    batch_pad = jnp.concatenate(
        [batch.astype(jnp.int32), jnp.full((_NPAD,), _G, jnp.int32)]
    ).reshape(1, npd)

    seg16 = _make_seg_sum(npd, 16, ec)
    seg32 = _make_seg_sum(npd, 32, ec)

    p1, r1 = _tc_call(
        _proj1_body,
        [jax.ShapeDtypeStruct((npd, 16), jnp.float32),
         jax.ShapeDtypeStruct((npd, 16), jnp.float32)],
        x, W1_rel, W1_root)
    acc1 = seg16(p1, src2d, dst2d, zeros16)
    h1 = _tc_call(
        _combine1_body, jax.ShapeDtypeStruct((npd, 16), jnp.float32),
        acc1, r1, b1_rel.reshape(1, 16))

    acc2 = seg16(h1, src2d, dst2d, zeros16)
    h2 = _tc_call(
        _combine_mm_body, jax.ShapeDtypeStruct((npd, 32), jnp.float32),
        acc2, h1, W2_rel, W2_root, b2_rel.reshape(1, 32))

    acc3 = seg32(h2, src2d, dst2d, zeros32)
    out = _tc_call(
        _final_body, jax.ShapeDtypeStruct((_G, d_in), jnp.float32),
        acc3, h2, W3_rel, W3_root, b3_rel.reshape(1, 32), batch_pad,
        W_lin, b_lin.reshape(1, d_in))
    return out


# speedup vs baseline: 22.8709x; 1.1829x over previous
import functools

import jax
import jax.numpy as jnp
from jax import lax
from jax.experimental import pallas as pl
from jax.experimental.pallas import tpu as pltpu
from jax.experimental.pallas import tpu_sc as plsc

_G = 64
_NPAD = 112
_CHUNK = 128
_NW = 32
_EALIGN = _NW * _CHUNK * 8


def _dotT(a, w):
    return lax.dot_general(a, w, (((1,), (1,)), ((), ())),
                           preferred_element_type=jnp.float32)


def _proj1_body(x_ref, wrel_ref, wroot_ref, p_ref, r_ref):
    n = x_ref.shape[0]
    x = x_ref[...]
    p_ref[0:n, :] = _dotT(x, wrel_ref[...])
    r_ref[0:n, :] = _dotT(x, wroot_ref[...])
    pad = p_ref.shape[0] - n
    p_ref[n:, :] = jnp.zeros((pad, p_ref.shape[1]), jnp.float32)
    r_ref[n:, :] = jnp.zeros((pad, r_ref.shape[1]), jnp.float32)


def _combine1_body(acc_ref, r_ref, b_ref, h_ref):
    n = r_ref.shape[0] - _NPAD
    h = jnp.maximum(acc_ref[0] + acc_ref[1] + r_ref[...] + b_ref[...], 0.0)
    h_ref[0:n, :] = h[0:n, :]
    h_ref[n:, :] = jnp.zeros((_NPAD, h_ref.shape[1]), jnp.float32)


def _combine_mm_body(acc_ref, h_ref, wrel_ref, wroot_ref, b_ref, out_ref):
    n = h_ref.shape[0] - _NPAD
    agg = acc_ref[0] + acc_ref[1]
    v = _dotT(agg, wrel_ref[...]) + _dotT(h_ref[...], wroot_ref[...])
    v = jnp.maximum(v + b_ref[...], 0.0)
    out_ref[0:n, :] = v[0:n, :]
    out_ref[n:, :] = jnp.zeros((_NPAD, out_ref.shape[1]), jnp.float32)


def _final_body(acc_ref, h_ref, wrel_ref, wroot_ref, b_ref, batch_ref,
                wlin_ref, blin_ref, out_ref):
    npd = h_ref.shape[0]
    agg = acc_ref[0] + acc_ref[1]
    h3 = _dotT(agg, wrel_ref[...]) + _dotT(h_ref[...], wroot_ref[...])
    h3 = jnp.maximum(h3 + b_ref[...], 0.0)
    seg = batch_ref[...]
    mt = (lax.broadcasted_iota(jnp.int32, (_G, npd), 0) == seg)
    pooled = lax.dot_general(mt.astype(jnp.float32), h3,
                             (((1,), (0,)), ((), ())),
                             preferred_element_type=jnp.float32)
    out_ref[...] = _dotT(pooled, wlin_ref[...]) + blin_ref[...]


def _tc_call(body, out_shapes, *args):
    return pl.pallas_call(
        body,
        out_shape=out_shapes,
    )(*args)


def _make_seg_sum(npd, d, ec):
    cpw = ec // _NW
    rpt = npd // 16
    k = 8
    ng = cpw // k
    mesh = plsc.VectorSubcoreMesh(core_axis_name="c", subcore_axis_name="s",
                                  num_cores=2, num_subcores=16)

    @functools.partial(
        pl.kernel,
        out_type=jax.ShapeDtypeStruct((2, npd, d), jnp.float32),
        mesh=mesh,
        compiler_params=pltpu.CompilerParams(use_tc_tiling_on_sc=False),
        scratch_types=[
            pltpu.VMEM((cpw, _CHUNK), jnp.int32),
            pltpu.VMEM((cpw, _CHUNK), jnp.int32),
            pltpu.VMEM((2, k, _CHUNK, d), jnp.float32),
            pltpu.VMEM_SHARED((npd, d), jnp.float32),
            pltpu.VMEM_SHARED((npd, d), jnp.float32),
            pltpu.SemaphoreType.DMA,
            pltpu.SemaphoreType.DMA,
            pltpu.SemaphoreType.DMA,
        ],
    )
    def seg(table, src, dst, zeros, out, sidx, didx, rows, tbl, acc,
            sem0, gsem, ssem):
        c = lax.axis_index("c")
        s = lax.axis_index("s")
        w = c * 16 + s
        cp1 = pltpu.async_copy(src.at[pl.ds(w * cpw, cpw)], sidx, sem0)
        cp2 = pltpu.async_copy(dst.at[pl.ds(w * cpw, cpw)], didx, sem0)
        cp3 = pltpu.async_copy(table.at[pl.ds(s * rpt, rpt)],
                               tbl.at[pl.ds(s * rpt, rpt)], sem0)
        cp4 = pltpu.async_copy(zeros.at[pl.ds(s * rpt, rpt)],
                               acc.at[pl.ds(s * rpt, rpt)], sem0)
        cp1.wait(); cp2.wait(); cp3.wait(); cp4.wait()
        plsc.subcore_barrier()

        def start_gathers(g, p):
            for b in range(k):
                pltpu.async_copy(tbl.at[sidx.at[g * k + b]],
                                 rows.at[p, b], gsem)

        start_gathers(0, 0)

        def body(g, carry):
            p = lax.rem(g, 2)
            for b in range(k):
                pltpu.make_async_copy(tbl.at[sidx.at[g * k + b]],
                                      rows.at[p, b], gsem).wait()
            @pl.when(g + 1 < ng)
            def _():
                start_gathers(g + 1, 1 - p)
            for b in range(k):
                pltpu.async_copy(rows.at[p, b], acc.at[didx.at[g * k + b]],
                                 ssem, add=True)
            for b in range(k):
                pltpu.make_async_copy(rows.at[p, b],
                                      acc.at[didx.at[g * k + b]],
                                      ssem).wait()
            return carry

        lax.fori_loop(0, ng, body, 0, unroll=False)
        plsc.subcore_barrier()
        pltpu.sync_copy(acc.at[pl.ds(s * rpt, rpt)],
                        out.at[c, pl.ds(s * rpt, rpt)])

    return seg


def kernel(x, edge_index, batch, W1_rel, b1_rel, W1_root, W2_rel, b2_rel,
           W2_root, W3_rel, b3_rel, W3_root, W_lin, b_lin):
    n, d_in = x.shape
    e = edge_index.shape[1]
    npd = n + _NPAD

    epad = -(-e // _EALIGN) * _EALIGN
    pad = epad - e
    src = jnp.concatenate([edge_index[0], jnp.full((pad,), n, jnp.int32)])
    dst = jnp.concatenate([edge_index[1], jnp.full((pad,), n, jnp.int32)])
    src2d = src.reshape(epad // _CHUNK, _CHUNK)
    dst2d = dst.reshape(epad // _CHUNK, _CHUNK)
    ec = epad // _CHUNK

    zeros16 = jnp.zeros((npd, 16), jnp.float32)
    zeros32 = jnp.zeros((npd, 32), jnp.float32)
    batch_pad = jnp.concatenate(
        [batch.astype(jnp.int32), jnp.full((_NPAD,), _G, jnp.int32)]
    ).reshape(1, npd)

    seg16 = _make_seg_sum(npd, 16, ec)
    seg32 = _make_seg_sum(npd, 32, ec)

    p1, r1 = _tc_call(
        _proj1_body,
        [jax.ShapeDtypeStruct((npd, 16), jnp.float32),
         jax.ShapeDtypeStruct((npd, 16), jnp.float32)],
        x, W1_rel, W1_root)
    acc1 = seg16(p1, src2d, dst2d, zeros16)
    h1 = _tc_call(
        _combine1_body, jax.ShapeDtypeStruct((npd, 16), jnp.float32),
        acc1, r1, b1_rel.reshape(1, 16))

    acc2 = seg16(h1, src2d, dst2d, zeros16)
    h2 = _tc_call(
        _combine_mm_body, jax.ShapeDtypeStruct((npd, 32), jnp.float32),
        acc2, h1, W2_rel, W2_root, b2_rel.reshape(1, 32))

    acc3 = seg32(h2, src2d, dst2d, zeros32)
    out = _tc_call(
        _final_body, jax.ShapeDtypeStruct((_G, d_in), jnp.float32),
        acc3, h2, W3_rel, W3_root, b3_rel.reshape(1, 32), batch_pad,
        W_lin, b_lin.reshape(1, d_in))
    return out


# speedup vs baseline: 24.0689x; 1.0524x over previous
import functools

import jax
import jax.numpy as jnp
from jax import lax
from jax.experimental import pallas as pl
from jax.experimental.pallas import tpu as pltpu
from jax.experimental.pallas import tpu_sc as plsc

_G = 64
_NPAD = 112
_CHUNK = 128
_NW = 32
_EALIGN = _NW * _CHUNK * 8


def _dotT(a, w):
    return lax.dot_general(a, w, (((1,), (1,)), ((), ())),
                           preferred_element_type=jnp.float32)


def _proj1_body(x_ref, wrel_ref, wroot_ref, p_ref, r_ref):
    n = x_ref.shape[0]
    x = x_ref[...]
    p_ref[0:n, :] = _dotT(x, wrel_ref[...])
    r_ref[0:n, :] = _dotT(x, wroot_ref[...])
    pad = p_ref.shape[0] - n
    p_ref[n:, :] = jnp.zeros((pad, p_ref.shape[1]), jnp.float32)
    r_ref[n:, :] = jnp.zeros((pad, r_ref.shape[1]), jnp.float32)


def _combine1_body(acc_ref, r_ref, b_ref, h_ref):
    n = r_ref.shape[0] - _NPAD
    h = jnp.maximum(acc_ref[0] + acc_ref[1] + r_ref[...] + b_ref[...], 0.0)
    h_ref[0:n, :] = h[0:n, :]
    h_ref[n:, :] = jnp.zeros((_NPAD, h_ref.shape[1]), jnp.float32)


def _combine_mm_body(acc_ref, h_ref, wrel_ref, wroot_ref, b_ref, out_ref):
    n = h_ref.shape[0] - _NPAD
    agg = acc_ref[0] + acc_ref[1]
    v = _dotT(agg, wrel_ref[...]) + _dotT(h_ref[...], wroot_ref[...])
    v = jnp.maximum(v + b_ref[...], 0.0)
    out_ref[0:n, :] = v[0:n, :]
    out_ref[n:, :] = jnp.zeros((_NPAD, out_ref.shape[1]), jnp.float32)


def _final_body(acc_ref, h_ref, wrel_ref, wroot_ref, b_ref, batch_ref,
                wlin_ref, blin_ref, out_ref):
    npd = h_ref.shape[0]
    agg = acc_ref[0] + acc_ref[1]
    h3 = _dotT(agg, wrel_ref[...]) + _dotT(h_ref[...], wroot_ref[...])
    h3 = jnp.maximum(h3 + b_ref[...], 0.0)
    seg = batch_ref[...]
    mt = (lax.broadcasted_iota(jnp.int32, (_G, npd), 0) == seg)
    pooled = lax.dot_general(mt.astype(jnp.float32), h3,
                             (((1,), (0,)), ((), ())),
                             preferred_element_type=jnp.float32)
    out_ref[...] = _dotT(pooled, wlin_ref[...]) + blin_ref[...]


def _tc_call(body, out_shapes, *args):
    return pl.pallas_call(
        body,
        out_shape=out_shapes,
    )(*args)


def _make_seg_sum(npd, d, ec):
    cpw = ec // _NW
    rpt = npd // 16
    k = 8
    ng = cpw // k
    mesh = plsc.VectorSubcoreMesh(core_axis_name="c", subcore_axis_name="s",
                                  num_cores=2, num_subcores=16)

    @functools.partial(
        pl.kernel,
        out_type=jax.ShapeDtypeStruct((2, npd, d), jnp.float32),
        mesh=mesh,
        compiler_params=pltpu.CompilerParams(use_tc_tiling_on_sc=False),
        scratch_types=[
            pltpu.VMEM((cpw, _CHUNK), jnp.int32),
            pltpu.VMEM((cpw, _CHUNK), jnp.int32),
            pltpu.VMEM((2, k, _CHUNK, d), jnp.float32),
            pltpu.VMEM_SHARED((npd, d), jnp.float32),
            pltpu.VMEM_SHARED((npd, d), jnp.float32),
            pltpu.SemaphoreType.DMA,
            pltpu.SemaphoreType.DMA,
            pltpu.SemaphoreType.DMA,
        ],
    )
    def seg(table, src, dst, zeros, out, sidx, didx, rows, tbl, acc,
            sem0, gsem, ssem):
        c = lax.axis_index("c")
        s = lax.axis_index("s")
        w = c * 16 + s
        cp1 = pltpu.async_copy(src.at[pl.ds(w * cpw, cpw)], sidx, sem0)
        cp2 = pltpu.async_copy(dst.at[pl.ds(w * cpw, cpw)], didx, sem0)
        cp3 = pltpu.async_copy(table.at[pl.ds(s * rpt, rpt)],
                               tbl.at[pl.ds(s * rpt, rpt)], sem0)
        cp4 = pltpu.async_copy(zeros.at[pl.ds(s * rpt, rpt)],
                               acc.at[pl.ds(s * rpt, rpt)], sem0)
        cp1.wait(); cp2.wait(); cp3.wait(); cp4.wait()
        plsc.subcore_barrier()

        def start_gathers(g, p):
            for b in range(k):
                pltpu.async_copy(tbl.at[sidx.at[g * k + b]],
                                 rows.at[p, b], gsem)

        start_gathers(0, 0)

        def body(g, carry):
            p = lax.rem(g, 2)
            for b in range(k):
                pltpu.make_async_copy(tbl.at[sidx.at[g * k + b]],
                                      rows.at[p, b], gsem).wait()
            @pl.when(g + 1 < ng)
            def _():
                start_gathers(g + 1, 1 - p)
            for b in range(k):
                pltpu.async_copy(rows.at[p, b], acc.at[didx.at[g * k + b]],
                                 ssem, add=True)
            for b in range(k):
                pltpu.make_async_copy(rows.at[p, b],
                                      acc.at[didx.at[g * k + b]],
                                      ssem).wait()
            return carry

        lax.fori_loop(0, ng, body, 0, unroll=False)
        plsc.subcore_barrier()
        pltpu.sync_copy(acc.at[pl.ds(s * rpt, rpt)],
                        out.at[c, pl.ds(s * rpt, rpt)])

    return seg


def _make_combine_seg_sum(npd, ec):
    d = 16
    cpw = ec // _NW
    rpt = npd // 16
    k = 8
    ng = cpw // k
    mesh = plsc.VectorSubcoreMesh(core_axis_name="c", subcore_axis_name="s",
                                  num_cores=2, num_subcores=16)

    @functools.partial(
        pl.kernel,
        out_type=(jax.ShapeDtypeStruct((2, npd, d), jnp.float32),
                  jax.ShapeDtypeStruct((npd, d), jnp.float32)),
        mesh=mesh,
        compiler_params=pltpu.CompilerParams(use_tc_tiling_on_sc=False),
        scratch_types=[
            pltpu.VMEM((cpw, _CHUNK), jnp.int32),
            pltpu.VMEM((cpw, _CHUNK), jnp.int32),
            pltpu.VMEM((2, k, _CHUNK, d), jnp.float32),
            pltpu.VMEM((rpt, d), jnp.float32),
            pltpu.VMEM((rpt, d), jnp.float32),
            pltpu.VMEM((rpt, d), jnp.float32),
            pltpu.VMEM((d,), jnp.float32),
            pltpu.VMEM_SHARED((npd, d), jnp.float32),
            pltpu.VMEM_SHARED((npd, d), jnp.float32),
            pltpu.SemaphoreType.DMA,
            pltpu.SemaphoreType.DMA,
            pltpu.SemaphoreType.DMA,
        ],
    )
    def seg(acc1, r1, b1, src, dst, zeros, out, h1_out, sidx, didx, rows,
            bufa, bufb, bufc, bvec, tbl, acc, sem0, gsem, ssem):
        c = lax.axis_index("c")
        s = lax.axis_index("s")
        w = c * 16 + s
        sl = pl.ds(s * rpt, rpt)
        cps = [
            pltpu.async_copy(src.at[pl.ds(w * cpw, cpw)], sidx, sem0),
            pltpu.async_copy(dst.at[pl.ds(w * cpw, cpw)], didx, sem0),
            pltpu.async_copy(acc1.at[0, sl], bufa, sem0),
            pltpu.async_copy(acc1.at[1, sl], bufb, sem0),
            pltpu.async_copy(r1.at[sl], bufc, sem0),
            pltpu.async_copy(b1, bvec, sem0),
            pltpu.async_copy(zeros.at[sl], acc.at[sl], sem0),
        ]
        for cp in cps:
            cp.wait()
        bv = bvec[...]

        def crow(i, carry):
            bufc[i] = jnp.maximum(bufa[i] + bufb[i] + bufc[i] + bv, 0.0)
            return carry

        lax.fori_loop(0, rpt, crow, 0, unroll=False)
        pltpu.sync_copy(bufc, tbl.at[sl])

        @pl.when(c == 0)
        def _():
            pltpu.sync_copy(bufc, h1_out.at[sl])

        plsc.subcore_barrier()

        def start_gathers(g, p):
            for b in range(k):
                pltpu.async_copy(tbl.at[sidx.at[g * k + b]],
                                 rows.at[p, b], gsem)

        start_gathers(0, 0)

        def body(g, carry):
            p = lax.rem(g, 2)
            for b in range(k):
                pltpu.make_async_copy(tbl.at[sidx.at[g * k + b]],
                                      rows.at[p, b], gsem).wait()

            @pl.when(g + 1 < ng)
            def _():
                start_gathers(g + 1, 1 - p)

            for b in range(k):
                pltpu.async_copy(rows.at[p, b], acc.at[didx.at[g * k + b]],
                                 ssem, add=True)
            for b in range(k):
                pltpu.make_async_copy(rows.at[p, b],
                                      acc.at[didx.at[g * k + b]],
                                      ssem).wait()
            return carry

        lax.fori_loop(0, ng, body, 0, unroll=False)
        plsc.subcore_barrier()
        pltpu.sync_copy(acc.at[sl], out.at[c, sl])

    return seg


def kernel(x, edge_index, batch, W1_rel, b1_rel, W1_root, W2_rel, b2_rel,
           W2_root, W3_rel, b3_rel, W3_root, W_lin, b_lin):
    n, d_in = x.shape
    e = edge_index.shape[1]
    npd = n + _NPAD

    epad = -(-e // _EALIGN) * _EALIGN
    pad = epad - e
    src = jnp.concatenate([edge_index[0], jnp.full((pad,), n, jnp.int32)])
    dst = jnp.concatenate([edge_index[1], jnp.full((pad,), n, jnp.int32)])
    src2d = src.reshape(epad // _CHUNK, _CHUNK)
    dst2d = dst.reshape(epad // _CHUNK, _CHUNK)
    ec = epad // _CHUNK

    zeros16 = jnp.zeros((npd, 16), jnp.float32)
    zeros32 = jnp.zeros((npd, 32), jnp.float32)
    batch_pad = jnp.concatenate(
        [batch.astype(jnp.int32), jnp.full((_NPAD,), _G, jnp.int32)]
    ).reshape(1, npd)

    seg16 = _make_seg_sum(npd, 16, ec)
    seg32 = _make_seg_sum(npd, 32, ec)
    seg2 = _make_combine_seg_sum(npd, ec)

    p1, r1 = _tc_call(
        _proj1_body,
        [jax.ShapeDtypeStruct((npd, 16), jnp.float32),
         jax.ShapeDtypeStruct((npd, 16), jnp.float32)],
        x, W1_rel, W1_root)
    acc1 = seg16(p1, src2d, dst2d, zeros16)

    acc2, h1 = seg2(acc1, r1, b1_rel, src2d, dst2d, zeros16)
    h2 = _tc_call(
        _combine_mm_body, jax.ShapeDtypeStruct((npd, 32), jnp.float32),
        acc2, h1, W2_rel, W2_root, b2_rel.reshape(1, 32))

    acc3 = seg32(h2, src2d, dst2d, zeros32)
    out = _tc_call(
        _final_body, jax.ShapeDtypeStruct((_G, d_in), jnp.float32),
        acc3, h2, W3_rel, W3_root, b3_rel.reshape(1, 32), batch_pad,
        W_lin, b_lin.reshape(1, d_in))
    return out


# speedup vs baseline: 25.5152x; 1.0601x over previous
import functools

import jax
import jax.numpy as jnp
from jax import lax
from jax.experimental import pallas as pl
from jax.experimental.pallas import tpu as pltpu
from jax.experimental.pallas import tpu_sc as plsc

_G = 64
_NPAD = 112
_CHUNK = 128
_NW = 32
_EALIGN = _NW * _CHUNK * 8


def _dotT(a, w):
    return lax.dot_general(a, w, (((1,), (1,)), ((), ())),
                           preferred_element_type=jnp.float32)


def _proj1_body(x_ref, wrel_ref, wroot_ref, p_ref, r_ref):
    n = x_ref.shape[0]
    x = x_ref[...]
    p_ref[0:n, :] = _dotT(x, wrel_ref[...])
    r_ref[0:n, :] = _dotT(x, wroot_ref[...])
    pad = p_ref.shape[0] - n
    p_ref[n:, :] = jnp.zeros((pad, p_ref.shape[1]), jnp.float32)
    r_ref[n:, :] = jnp.zeros((pad, r_ref.shape[1]), jnp.float32)


def _combine_mm_body(acc_ref, h_ref, wrel_ref, wroot_ref, b_ref, out_ref,
                     *, n, din, dout):
    agg = acc_ref[0] + acc_ref[1]
    h = h_ref[...]
    v = _dotT(agg, wrel_ref[...]) + _dotT(h, wroot_ref[...])
    v = jnp.maximum(v + b_ref[...], 0.0)
    out_ref[0:n, :] = v[0:n, :]
    out_ref[n:, :] = jnp.zeros((out_ref.shape[0] - n, dout), jnp.float32)


def _final_body(acc_ref, h_ref, wrel_ref, wroot_ref, b_ref, batch_ref,
                wlin_ref, blin_ref, out_ref):
    npd = acc_ref.shape[1]
    agg = acc_ref[0] + acc_ref[1]
    h = h_ref[...]
    h3 = _dotT(agg, wrel_ref[...]) + _dotT(h, wroot_ref[...])
    h3 = jnp.maximum(h3 + b_ref[...], 0.0)
    seg = batch_ref[...]
    mt = (lax.broadcasted_iota(jnp.int32, (_G, npd), 0) == seg)
    pooled = lax.dot_general(mt.astype(jnp.float32), h3,
                             (((1,), (0,)), ((), ())),
                             preferred_element_type=jnp.float32)
    out_ref[...] = _dotT(pooled, wlin_ref[...]) + blin_ref[...]


def _tc_call(body, out_shapes, *args):
    return pl.pallas_call(
        body,
        out_shape=out_shapes,
    )(*args)


def _make_seg_sum(npd, d, ec):
    cpw = ec // _NW
    rpt = npd // 16
    k = 8
    ng = cpw // k
    mesh = plsc.VectorSubcoreMesh(core_axis_name="c", subcore_axis_name="s",
                                  num_cores=2, num_subcores=16)

    @functools.partial(
        pl.kernel,
        out_type=jax.ShapeDtypeStruct((2, npd, d), jnp.float32),
        mesh=mesh,
        compiler_params=pltpu.CompilerParams(use_tc_tiling_on_sc=False),
        scratch_types=[
            pltpu.VMEM((cpw, _CHUNK), jnp.int32),
            pltpu.VMEM((cpw, _CHUNK), jnp.int32),
            pltpu.VMEM((2, k, _CHUNK, d), jnp.float32),
            pltpu.VMEM_SHARED((npd, d), jnp.float32),
            pltpu.VMEM_SHARED((npd, d), jnp.float32),
            pltpu.SemaphoreType.DMA,
            pltpu.SemaphoreType.DMA,
            pltpu.SemaphoreType.DMA,
        ],
    )
    def seg(table, ei, zeros, out, sidx, didx, rows, tbl, acc,
            sem0, gsem, ssem):
        c = lax.axis_index("c")
        s = lax.axis_index("s")
        w = c * 16 + s
        cp1 = pltpu.async_copy(ei.at[0, pl.ds(w * cpw, cpw)], sidx, sem0)
        cp2 = pltpu.async_copy(ei.at[1, pl.ds(w * cpw, cpw)], didx, sem0)
        cp3 = pltpu.async_copy(table.at[pl.ds(s * rpt, rpt)],
                               tbl.at[pl.ds(s * rpt, rpt)], sem0)
        cp4 = pltpu.async_copy(zeros.at[pl.ds(s * rpt, rpt)],
                               acc.at[pl.ds(s * rpt, rpt)], sem0)
        cp1.wait(); cp2.wait(); cp3.wait(); cp4.wait()
        plsc.subcore_barrier()

        def start_gathers(g, p):
            for b in range(k):
                pltpu.async_copy(tbl.at[sidx.at[g * k + b]],
                                 rows.at[p, b], gsem)

        start_gathers(0, 0)

        def body(g, carry):
            p = lax.rem(g, 2)
            for b in range(k):
                pltpu.make_async_copy(tbl.at[sidx.at[g * k + b]],
                                      rows.at[p, b], gsem).wait()
            @pl.when(g + 1 < ng)
            def _():
                start_gathers(g + 1, 1 - p)
            for b in range(k):
                pltpu.async_copy(rows.at[p, b], acc.at[didx.at[g * k + b]],
                                 ssem, add=True)
            for b in range(k):
                pltpu.make_async_copy(rows.at[p, b],
                                      acc.at[didx.at[g * k + b]],
                                      ssem).wait()
            return carry

        lax.fori_loop(0, ng, body, 0, unroll=False)
        plsc.subcore_barrier()
        pltpu.sync_copy(acc.at[pl.ds(s * rpt, rpt)],
                        out.at[c, pl.ds(s * rpt, rpt)])

    return seg


def _make_combine_seg_sum(npd, ec):
    d = 16
    cpw = ec // _NW
    rpt = npd // 16
    k = 8
    ng = cpw // k
    mesh = plsc.VectorSubcoreMesh(core_axis_name="c", subcore_axis_name="s",
                                  num_cores=2, num_subcores=16)

    @functools.partial(
        pl.kernel,
        out_type=(jax.ShapeDtypeStruct((2, npd, d), jnp.float32),
                  jax.ShapeDtypeStruct((npd, d), jnp.float32)),
        mesh=mesh,
        compiler_params=pltpu.CompilerParams(use_tc_tiling_on_sc=False),
        scratch_types=[
            pltpu.VMEM((cpw, _CHUNK), jnp.int32),
            pltpu.VMEM((cpw, _CHUNK), jnp.int32),
            pltpu.VMEM((2, k, _CHUNK, d), jnp.float32),
            pltpu.VMEM((rpt, d), jnp.float32),
            pltpu.VMEM((rpt, d), jnp.float32),
            pltpu.VMEM((rpt, d), jnp.float32),
            pltpu.VMEM((d,), jnp.float32),
            pltpu.VMEM_SHARED((npd, d), jnp.float32),
            pltpu.VMEM_SHARED((npd, d), jnp.float32),
            pltpu.SemaphoreType.DMA,
            pltpu.SemaphoreType.DMA,
            pltpu.SemaphoreType.DMA,
        ],
    )
    def seg(acc1, r1, b1, ei, zeros, out, h1_out, sidx, didx, rows,
            bufa, bufb, bufc, bvec, tbl, acc, sem0, gsem, ssem):
        c = lax.axis_index("c")
        s = lax.axis_index("s")
        w = c * 16 + s
        sl = pl.ds(s * rpt, rpt)
        cps = [
            pltpu.async_copy(ei.at[0, pl.ds(w * cpw, cpw)], sidx, sem0),
            pltpu.async_copy(ei.at[1, pl.ds(w * cpw, cpw)], didx, sem0),
            pltpu.async_copy(acc1.at[0, sl], bufa, sem0),
            pltpu.async_copy(acc1.at[1, sl], bufb, sem0),
            pltpu.async_copy(r1.at[sl], bufc, sem0),
            pltpu.async_copy(b1, bvec, sem0),
            pltpu.async_copy(zeros.at[sl], acc.at[sl], sem0),
        ]
        for cp in cps:
            cp.wait()
        bv = bvec[...]

        def crow(i, carry):
            bufc[i] = jnp.maximum(bufa[i] + bufb[i] + bufc[i] + bv, 0.0)
            return carry

        lax.fori_loop(0, rpt, crow, 0, unroll=False)
        pltpu.sync_copy(bufc, tbl.at[sl])

        @pl.when(c == 0)
        def _():
            pltpu.sync_copy(bufc, h1_out.at[sl])

        plsc.subcore_barrier()

        def start_gathers(g, p):
            for b in range(k):
                pltpu.async_copy(tbl.at[sidx.at[g * k + b]],
                                 rows.at[p, b], gsem)

        start_gathers(0, 0)

        def body(g, carry):
            p = lax.rem(g, 2)
            for b in range(k):
                pltpu.make_async_copy(tbl.at[sidx.at[g * k + b]],
                                      rows.at[p, b], gsem).wait()

            @pl.when(g + 1 < ng)
            def _():
                start_gathers(g + 1, 1 - p)

            for b in range(k):
                pltpu.async_copy(rows.at[p, b], acc.at[didx.at[g * k + b]],
                                 ssem, add=True)
            for b in range(k):
                pltpu.make_async_copy(rows.at[p, b],
                                      acc.at[didx.at[g * k + b]],
                                      ssem).wait()
            return carry

        lax.fori_loop(0, ng, body, 0, unroll=False)
        plsc.subcore_barrier()
        pltpu.sync_copy(acc.at[sl], out.at[c, sl])

    return seg


def kernel(x, edge_index, batch, W1_rel, b1_rel, W1_root, W2_rel, b2_rel,
           W2_root, W3_rel, b3_rel, W3_root, W_lin, b_lin):
    n, d_in = x.shape
    e = edge_index.shape[1]
    npd = n + _NPAD

    epad = -(-e // _EALIGN) * _EALIGN
    ec = epad // _CHUNK
    ei = jnp.pad(edge_index, ((0, 0), (0, epad - e)),
                 constant_values=n).reshape(2, ec, _CHUNK)

    zeros16 = jnp.zeros((npd, 16), jnp.float32)
    zeros32 = jnp.zeros((npd, 32), jnp.float32)
    batch_pad = jnp.concatenate(
        [batch.astype(jnp.int32), jnp.full((_NPAD,), _G, jnp.int32)]
    ).reshape(1, npd)

    seg16 = _make_seg_sum(npd, 16, ec)
    seg32 = _make_seg_sum(npd, 32, ec)
    seg2 = _make_combine_seg_sum(npd, ec)

    pk16 = npd * 16 // 128
    pk32 = npd * 32 // 128

    p1, r1 = _tc_call(
        _proj1_body,
        [jax.ShapeDtypeStruct((npd, 16), jnp.float32),
         jax.ShapeDtypeStruct((npd, 16), jnp.float32)],
        x, W1_rel, W1_root)
    acc1 = seg16(p1, ei, zeros16)

    acc2, h1 = seg2(acc1, r1, b1_rel, ei, zeros16)

    h2 = _tc_call(
        functools.partial(_combine_mm_body, n=n, din=16, dout=32),
        jax.ShapeDtypeStruct((npd, 32), jnp.float32),
        acc2, h1, W2_rel, W2_root, b2_rel.reshape(1, 32))

    acc3 = seg32(h2, ei, zeros32)
    out = _tc_call(
        _final_body, jax.ShapeDtypeStruct((_G, d_in), jnp.float32),
        acc3, h2, W3_rel, W3_root,
        b3_rel.reshape(1, 32), batch_pad, W_lin, b_lin.reshape(1, d_in))
    return out


# speedup vs baseline: 29.9259x; 1.1729x over previous
import functools

import jax
import jax.numpy as jnp
from jax import lax
from jax.experimental import pallas as pl
from jax.experimental.pallas import tpu as pltpu
from jax.experimental.pallas import tpu_sc as plsc

_G = 64
_NPAD = 240
_CHUNK = 128
_NW = 32
_EALIGN = _NW * _CHUNK * 8


def _dotT(a, w):
    return lax.dot_general(a, w, (((1,), (1,)), ((), ())),
                           preferred_element_type=jnp.float32)


def _proj1_body(x_ref, wrel_ref, wroot_ref, p_ref, r_ref):
    n = x_ref.shape[0]
    x = x_ref[...]
    p_ref[0:n, :] = _dotT(x, wrel_ref[...])
    r_ref[0:n, :] = _dotT(x, wroot_ref[...])
    pad = p_ref.shape[0] - n
    p_ref[n:, :] = jnp.zeros((pad, p_ref.shape[1]), jnp.float32)
    r_ref[n:, :] = jnp.zeros((pad, r_ref.shape[1]), jnp.float32)


def _combine_mm_body(acc_ref, h_ref, wra_ref, wta_ref, wrb_ref, wtb_ref,
                     ba_ref, bb_ref, outa_ref, outb_ref, *, npk):
    agg = acc_ref[0] + acc_ref[1]
    h = h_ref[...]
    pad = outa_ref.shape[0] - npk
    z = jnp.zeros((pad, 128), jnp.float32)
    va = jnp.maximum(agg @ wra_ref[...] + h @ wta_ref[...] + ba_ref[...], 0.0)
    outa_ref[0:npk, :] = va[0:npk, :]
    outa_ref[npk:, :] = z
    vb = jnp.maximum(agg @ wrb_ref[...] + h @ wtb_ref[...] + bb_ref[...], 0.0)
    outb_ref[0:npk, :] = vb[0:npk, :]
    outb_ref[npk:, :] = z


def _final_body(acca_ref, accb_ref, ha_ref, hb_ref, wstack_ref, ba_ref,
                bb_ref, bcols_ref, wlin_ref, blin_ref, out_ref):
    pk = ha_ref.shape[0]
    agg_a = acca_ref[0] + acca_ref[1]
    agg_b = accb_ref[0] + accb_ref[1]
    h2a = ha_ref[...]
    h2b = hb_ref[...]
    h3a = jnp.maximum(agg_a @ wstack_ref[0] + agg_b @ wstack_ref[1]
                      + h2a @ wstack_ref[2] + h2b @ wstack_ref[3]
                      + ba_ref[...], 0.0)
    h3b = jnp.maximum(agg_a @ wstack_ref[4] + agg_b @ wstack_ref[5]
                      + h2a @ wstack_ref[6] + h2b @ wstack_ref[7]
                      + bb_ref[...], 0.0)
    pooled_a = jnp.zeros((_G, 16), jnp.float32)
    pooled_b = jnp.zeros((_G, 16), jnp.float32)
    for i in range(8):
        seg = bcols_ref[i:i + 1, :]
        mt = (lax.broadcasted_iota(jnp.int32, (_G, pk), 0) == seg)
        mtf = mt.astype(jnp.float32)
        pa = lax.dot_general(mtf, h3a, (((1,), (0,)), ((), ())),
                             preferred_element_type=jnp.float32)
        pb = lax.dot_general(mtf, h3b, (((1,), (0,)), ((), ())),
                             preferred_element_type=jnp.float32)
        pooled_a = pooled_a + pa[:, 16 * i:16 * (i + 1)]
        pooled_b = pooled_b + pb[:, 16 * i:16 * (i + 1)]
    pooled = jnp.concatenate([pooled_a, pooled_b], axis=1)
    out_ref[...] = _dotT(pooled, wlin_ref[...]) + blin_ref[...]


def _tc_call(body, out_shapes, *args):
    return pl.pallas_call(
        body,
        out_shape=out_shapes,
    )(*args)


def _make_seg_sum(npd, d, ec):
    cpw = ec // _NW
    rpt = npd // 16
    k = 8
    ng = cpw // k
    mesh = plsc.VectorSubcoreMesh(core_axis_name="c", subcore_axis_name="s",
                                  num_cores=2, num_subcores=16)

    @functools.partial(
        pl.kernel,
        out_type=jax.ShapeDtypeStruct((2, npd, d), jnp.float32),
        mesh=mesh,
        compiler_params=pltpu.CompilerParams(use_tc_tiling_on_sc=False),
        scratch_types=[
            pltpu.VMEM((cpw, _CHUNK), jnp.int32),
            pltpu.VMEM((cpw, _CHUNK), jnp.int32),
            pltpu.VMEM((2, k, _CHUNK, d), jnp.float32),
            pltpu.VMEM_SHARED((npd, d), jnp.float32),
            pltpu.VMEM_SHARED((npd, d), jnp.float32),
            pltpu.SemaphoreType.DMA,
            pltpu.SemaphoreType.DMA,
            pltpu.SemaphoreType.DMA,
        ],
    )
    def seg(table, ei, zeros, out, sidx, didx, rows, tbl, acc,
            sem0, gsem, ssem):
        c = lax.axis_index("c")
        s = lax.axis_index("s")
        w = c * 16 + s
        cp1 = pltpu.async_copy(ei.at[0, pl.ds(w * cpw, cpw)], sidx, sem0)
        cp2 = pltpu.async_copy(ei.at[1, pl.ds(w * cpw, cpw)], didx, sem0)
        cp3 = pltpu.async_copy(table.at[pl.ds(s * rpt, rpt)],
                               tbl.at[pl.ds(s * rpt, rpt)], sem0)
        cp4 = pltpu.async_copy(zeros.at[pl.ds(s * rpt, rpt)],
                               acc.at[pl.ds(s * rpt, rpt)], sem0)
        cp1.wait(); cp2.wait(); cp3.wait(); cp4.wait()
        plsc.subcore_barrier()

        def start_gathers(g, p):
            for b in range(k):
                pltpu.async_copy(tbl.at[sidx.at[g * k + b]],
                                 rows.at[p, b], gsem)

        start_gathers(0, 0)

        def body(g, carry):
            p = lax.rem(g, 2)
            for b in range(k):
                pltpu.make_async_copy(tbl.at[sidx.at[g * k + b]],
                                      rows.at[p, b], gsem).wait()
            @pl.when(g + 1 < ng)
            def _():
                start_gathers(g + 1, 1 - p)
            for b in range(k):
                pltpu.async_copy(rows.at[p, b], acc.at[didx.at[g * k + b]],
                                 ssem, add=True)
            for b in range(k):
                pltpu.make_async_copy(rows.at[p, b],
                                      acc.at[didx.at[g * k + b]],
                                      ssem).wait()
            return carry

        lax.fori_loop(0, ng, body, 0, unroll=False)
        plsc.subcore_barrier()
        pltpu.sync_copy(acc.at[pl.ds(s * rpt, rpt)],
                        out.at[c, pl.ds(s * rpt, rpt)])

    return seg


def _make_combine_seg_sum(npd, ec):
    d = 16
    cpw = ec // _NW
    rpt = npd // 16
    k = 8
    ng = cpw // k
    mesh = plsc.VectorSubcoreMesh(core_axis_name="c", subcore_axis_name="s",
                                  num_cores=2, num_subcores=16)

    @functools.partial(
        pl.kernel,
        out_type=(jax.ShapeDtypeStruct((2, npd, d), jnp.float32),
                  jax.ShapeDtypeStruct((npd, d), jnp.float32)),
        mesh=mesh,
        compiler_params=pltpu.CompilerParams(use_tc_tiling_on_sc=False),
        scratch_types=[
            pltpu.VMEM((cpw, _CHUNK), jnp.int32),
            pltpu.VMEM((cpw, _CHUNK), jnp.int32),
            pltpu.VMEM((2, k, _CHUNK, d), jnp.float32),
            pltpu.VMEM((rpt, d), jnp.float32),
            pltpu.VMEM((rpt, d), jnp.float32),
            pltpu.VMEM((rpt, d), jnp.float32),
            pltpu.VMEM((d,), jnp.float32),
            pltpu.VMEM_SHARED((npd, d), jnp.float32),
            pltpu.VMEM_SHARED((npd, d), jnp.float32),
            pltpu.SemaphoreType.DMA,
            pltpu.SemaphoreType.DMA,
            pltpu.SemaphoreType.DMA,
        ],
    )
    def seg(acc1, r1, b1, ei, zeros, out, h1_out, sidx, didx, rows,
            bufa, bufb, bufc, bvec, tbl, acc, sem0, gsem, ssem):
        c = lax.axis_index("c")
        s = lax.axis_index("s")
        w = c * 16 + s
        sl = pl.ds(s * rpt, rpt)
        cps = [
            pltpu.async_copy(ei.at[0, pl.ds(w * cpw, cpw)], sidx, sem0),
            pltpu.async_copy(ei.at[1, pl.ds(w * cpw, cpw)], didx, sem0),
            pltpu.async_copy(acc1.at[0, sl], bufa, sem0),
            pltpu.async_copy(acc1.at[1, sl], bufb, sem0),
            pltpu.async_copy(r1.at[sl], bufc, sem0),
            pltpu.async_copy(b1, bvec, sem0),
            pltpu.async_copy(zeros.at[sl], acc.at[sl], sem0),
        ]
        for cp in cps:
            cp.wait()
        bv = bvec[...]

        def crow(i, carry):
            bufc[i] = jnp.maximum(bufa[i] + bufb[i] + bufc[i] + bv, 0.0)
            return carry

        lax.fori_loop(0, rpt, crow, 0, unroll=False)
        pltpu.sync_copy(bufc, tbl.at[sl])

        @pl.when(c == 0)
        def _():
            pltpu.sync_copy(bufc, h1_out.at[sl])

        plsc.subcore_barrier()

        def start_gathers(g, p):
            for b in range(k):
                pltpu.async_copy(tbl.at[sidx.at[g * k + b]],
                                 rows.at[p, b], gsem)

        start_gathers(0, 0)

        def body(g, carry):
            p = lax.rem(g, 2)
            for b in range(k):
                pltpu.make_async_copy(tbl.at[sidx.at[g * k + b]],
                                      rows.at[p, b], gsem).wait()

            @pl.when(g + 1 < ng)
            def _():
                start_gathers(g + 1, 1 - p)

            for b in range(k):
                pltpu.async_copy(rows.at[p, b], acc.at[didx.at[g * k + b]],
                                 ssem, add=True)
            for b in range(k):
                pltpu.make_async_copy(rows.at[p, b],
                                      acc.at[didx.at[g * k + b]],
                                      ssem).wait()
            return carry

        lax.fori_loop(0, ng, body, 0, unroll=False)
        plsc.subcore_barrier()
        pltpu.sync_copy(acc.at[sl], out.at[c, sl])

    return seg


def _make_seg_sum_pair(npd, ec):
    d = 16
    cpw = ec // _NW
    rpt = npd // 16
    k = 4
    ng = cpw // k
    mesh = plsc.VectorSubcoreMesh(core_axis_name="c", subcore_axis_name="s",
                                  num_cores=2, num_subcores=16)

    @functools.partial(
        pl.kernel,
        out_type=(jax.ShapeDtypeStruct((2, npd, d), jnp.float32),
                  jax.ShapeDtypeStruct((2, npd, d), jnp.float32)),
        mesh=mesh,
        compiler_params=pltpu.CompilerParams(use_tc_tiling_on_sc=False),
        scratch_types=[
            pltpu.VMEM((cpw, _CHUNK), jnp.int32),
            pltpu.VMEM((cpw, _CHUNK), jnp.int32),
            pltpu.VMEM((2, k, _CHUNK, d), jnp.float32),
            pltpu.VMEM((2, k, _CHUNK, d), jnp.float32),
            pltpu.VMEM_SHARED((npd, d), jnp.float32),
            pltpu.VMEM_SHARED((npd, d), jnp.float32),
            pltpu.VMEM_SHARED((npd, d), jnp.float32),
            pltpu.VMEM_SHARED((npd, d), jnp.float32),
            pltpu.SemaphoreType.DMA,
            pltpu.SemaphoreType.DMA,
            pltpu.SemaphoreType.DMA,
        ],
    )
    def seg(ta_hbm, tb_hbm, ei, zeros, outa, outb, sidx, didx, rowsa, rowsb,
            tba, tbb, acca, accb, sem0, gsem, ssem):
        c = lax.axis_index("c")
        s = lax.axis_index("s")
        w = c * 16 + s
        sl = pl.ds(s * rpt, rpt)
        cps = [
            pltpu.async_copy(ei.at[0, pl.ds(w * cpw, cpw)], sidx, sem0),
            pltpu.async_copy(ei.at[1, pl.ds(w * cpw, cpw)], didx, sem0),
            pltpu.async_copy(ta_hbm.at[sl], tba.at[sl], sem0),
            pltpu.async_copy(tb_hbm.at[sl], tbb.at[sl], sem0),
            pltpu.async_copy(zeros.at[sl], acca.at[sl], sem0),
            pltpu.async_copy(zeros.at[sl], accb.at[sl], sem0),
        ]
        for cp in cps:
            cp.wait()
        plsc.subcore_barrier()

        def start_gathers(g, p):
            for b in range(k):
                pltpu.async_copy(tba.at[sidx.at[g * k + b]],
                                 rowsa.at[p, b], gsem)
                pltpu.async_copy(tbb.at[sidx.at[g * k + b]],
                                 rowsb.at[p, b], gsem)

        start_gathers(0, 0)

        def body(g, carry):
            p = lax.rem(g, 2)
            for b in range(k):
                pltpu.make_async_copy(tba.at[sidx.at[g * k + b]],
                                      rowsa.at[p, b], gsem).wait()
                pltpu.make_async_copy(tbb.at[sidx.at[g * k + b]],
                                      rowsb.at[p, b], gsem).wait()

            @pl.when(g + 1 < ng)
            def _():
                start_gathers(g + 1, 1 - p)

            for b in range(k):
                pltpu.async_copy(rowsa.at[p, b], acca.at[didx.at[g * k + b]],
                                 ssem, add=True)
                pltpu.async_copy(rowsb.at[p, b], accb.at[didx.at[g * k + b]],
                                 ssem, add=True)
            for b in range(k):
                pltpu.make_async_copy(rowsa.at[p, b],
                                      acca.at[didx.at[g * k + b]],
                                      ssem).wait()
                pltpu.make_async_copy(rowsb.at[p, b],
                                      accb.at[didx.at[g * k + b]],
                                      ssem).wait()
            return carry

        lax.fori_loop(0, ng, body, 0, unroll=False)
        plsc.subcore_barrier()
        pltpu.sync_copy(acca.at[sl], outa.at[c, sl])
        pltpu.sync_copy(accb.at[sl], outb.at[c, sl])

    return seg


def kernel(x, edge_index, batch, W1_rel, b1_rel, W1_root, W2_rel, b2_rel,
           W2_root, W3_rel, b3_rel, W3_root, W_lin, b_lin):
    n, d_in = x.shape
    e = edge_index.shape[1]
    npd = n + _NPAD

    epad = -(-e // _EALIGN) * _EALIGN
    ec = epad // _CHUNK
    ei = jnp.pad(edge_index, ((0, 0), (0, epad - e)),
                 constant_values=n).reshape(2, ec, _CHUNK)

    zeros16 = jnp.zeros((npd, 16), jnp.float32)
    batch_cols = jnp.pad(batch.astype(jnp.int32), (0, _NPAD),
                         constant_values=_G).reshape(npd // 8, 8).T

    seg16 = _make_seg_sum(npd, 16, ec)
    seg2 = _make_combine_seg_sum(npd, ec)
    seg3 = _make_seg_sum_pair(npd, ec)

    pk16 = npd * 16 // 128
    npk = n * 16 // 128

    eye8 = jnp.eye(8, dtype=jnp.float32)

    def _kr(half):
        return jnp.kron(eye8, half)

    def _tile8(b):
        return jnp.tile(b, 8).reshape(1, 128)

    p1, r1 = _tc_call(
        _proj1_body,
        [jax.ShapeDtypeStruct((npd, 16), jnp.float32),
         jax.ShapeDtypeStruct((npd, 16), jnp.float32)],
        x, W1_rel, W1_root)
    acc1 = seg16(p1, ei, zeros16)

    acc2, h1 = seg2(acc1, r1, b1_rel, ei, zeros16)

    w2rT = W2_rel.T
    w2tT = W2_root.T
    h2a, h2b = _tc_call(
        functools.partial(_combine_mm_body, npk=npk),
        [jax.ShapeDtypeStruct((pk16, 128), jnp.float32),
         jax.ShapeDtypeStruct((pk16, 128), jnp.float32)],
        acc2.reshape(2, pk16, 128), h1.reshape(pk16, 128),
        _kr(w2rT[:, 0:16]), _kr(w2tT[:, 0:16]),
        _kr(w2rT[:, 16:32]), _kr(w2tT[:, 16:32]),
        _tile8(b2_rel[0:16]), _tile8(b2_rel[16:32]))

    acc3a, acc3b = seg3(h2a.reshape(npd, 16), h2b.reshape(npd, 16),
                        ei, zeros16)
    w3rT = W3_rel.T
    w3tT = W3_root.T
    wstack = jnp.stack([
        _kr(w3rT[0:16, 0:16]), _kr(w3rT[16:32, 0:16]),
        _kr(w3tT[0:16, 0:16]), _kr(w3tT[16:32, 0:16]),
        _kr(w3rT[0:16, 16:32]), _kr(w3rT[16:32, 16:32]),
        _kr(w3tT[0:16, 16:32]), _kr(w3tT[16:32, 16:32])])
    out = _tc_call(
        _final_body, jax.ShapeDtypeStruct((_G, d_in), jnp.float32),
        acc3a.reshape(2, pk16, 128), acc3b.reshape(2, pk16, 128),
        h2a, h2b, wstack, _tile8(b3_rel[0:16]), _tile8(b3_rel[16:32]),
        batch_cols, W_lin, b_lin.reshape(1, d_in))
    return out
